# Initial kernel scaffold; baseline (speedup 1.0000x reference)
import jax
import jax.numpy as jnp
from jax.experimental import pallas as pl


def kernel(x, edge_index, lin_w, lin_b, conv_w1, conv_w2, bn_gamma, bn_beta):
    raise NotImplementedError("write your pallas kernel here")


# speedup vs baseline: 4.5405x; 4.5405x over previous
import functools
import math

import jax
import jax.numpy as jnp
from jax import lax
from jax.experimental import pallas as pl
from jax.experimental.pallas import tpu as pltpu
from jax.experimental.pallas import tpu_sc as plsc

N = 10000
F = 128
E = 320000
_NPAD = 10240
ALPHA = 0.1
THETA = 0.5
EPS = 1e-5

_ROW_BLOCK = 1000
_GRID = N // _ROW_BLOCK


def _x0_body(x_ref, w_ref, b_ref, o_ref):
  acc = lax.dot_general(x_ref[...], w_ref[...], (((1,), (1,)), ((), ())),
                        preferred_element_type=jnp.float32)
  o_ref[...] = jnp.maximum(acc + b_ref[...], 0.0)


def _compute_x0(x, lin_w, lin_b2):
  return pl.pallas_call(
      _x0_body,
      grid=(_GRID,),
      in_specs=[
          pl.BlockSpec((_ROW_BLOCK, F), lambda i: (i, 0)),
          pl.BlockSpec((F, F), lambda i: (0, 0)),
          pl.BlockSpec((1, F), lambda i: (0, 0)),
      ],
      out_specs=pl.BlockSpec((_ROW_BLOCK, F), lambda i: (i, 0)),
      out_shape=jax.ShapeDtypeStruct((N, F), jnp.float32),
  )(x, lin_w, lin_b2)


def _seg_dense_body(m_ref, x0_ref, w_ref, h_ref, s_ref, *, beta):
  m = m_ref[0] + m_ref[1]
  t = (1.0 - ALPHA) * m + ALPHA * x0_ref[...]
  h = (1.0 - beta) * t + beta * jnp.dot(
      t, w_ref[...], preferred_element_type=jnp.float32)
  h_ref[...] = h

  @pl.when(pl.program_id(0) == 0)
  def _():
    s_ref[...] = jnp.zeros_like(s_ref)

  s_ref[0:1, :] = s_ref[0:1, :] + jnp.sum(h, axis=0, keepdims=True)
  s_ref[1:2, :] = s_ref[1:2, :] + jnp.sum(h * h, axis=0, keepdims=True)


def _bn_body(h_ref, s_ref, g_ref, b_ref, o_ref):
  h = h_ref[...]
  mean = s_ref[0:1, :] * (1.0 / N)
  var = s_ref[1:2, :] * (1.0 / N) - mean * mean
  inv = lax.rsqrt(var + EPS)
  o_ref[...] = jnp.maximum((h - mean) * inv * g_ref[...] + b_ref[...], 0.0)


def _dense_layer(mpart, x0, w, g2, b2, beta):
  h, stats = pl.pallas_call(
      functools.partial(_seg_dense_body, beta=beta),
      grid=(_GRID,),
      in_specs=[
          pl.BlockSpec((2, _ROW_BLOCK, F), lambda i: (0, i, 0)),
          pl.BlockSpec((_ROW_BLOCK, F), lambda i: (i, 0)),
          pl.BlockSpec((F, F), lambda i: (0, 0)),
      ],
      out_specs=[
          pl.BlockSpec((_ROW_BLOCK, F), lambda i: (i, 0)),
          pl.BlockSpec((8, F), lambda i: (0, 0)),
      ],
      out_shape=[
          jax.ShapeDtypeStruct((N, F), jnp.float32),
          jax.ShapeDtypeStruct((8, F), jnp.float32),
      ],
  )(mpart, x0, w)
  return pl.pallas_call(
      _bn_body,
      grid=(_GRID,),
      in_specs=[
          pl.BlockSpec((_ROW_BLOCK, F), lambda i: (i, 0)),
          pl.BlockSpec((8, F), lambda i: (0, 0)),
          pl.BlockSpec((1, F), lambda i: (0, 0)),
          pl.BlockSpec((1, F), lambda i: (0, 0)),
      ],
      out_specs=pl.BlockSpec((_ROW_BLOCK, F), lambda i: (i, 0)),
      out_shape=jax.ShapeDtypeStruct((N, F), jnp.float32),
  )(h, stats, g2, b2)


def _make_segsum():
  info = plsc.get_sparse_core_info()
  nc, ns = info.num_cores, info.num_subcores
  nw = nc * ns
  chunk = 80
  epw = E // nw
  niter = epw // chunk
  npad = _NPAD
  rpt = npad // ns
  zr = 128

  mesh = plsc.VectorSubcoreMesh(core_axis_name="c", subcore_axis_name="s")

  @functools.partial(
      pl.kernel,
      out_type=jax.ShapeDtypeStruct((nc, npad, F), jnp.float32),
      mesh=mesh,
      scratch_types=[
          pltpu.VMEM((chunk,), jnp.int32),
          pltpu.VMEM((chunk,), jnp.int32),
          pltpu.VMEM((chunk, F), jnp.float32),
          pltpu.VMEM((zr, F), jnp.float32),
          pltpu.VMEM_SHARED((npad, F), jnp.float32),
          pltpu.SemaphoreType.DMA,
      ],
  )
  def segsum(h_hbm, src_hbm, dst_hbm, out_hbm, sidx, didx, rows, zbuf, acc,
             sem):
    cid = lax.axis_index("c")
    sid = lax.axis_index("s")
    wid = sid * nc + cid

    def zero_body(i, carry):
      zbuf[i // 8, pl.ds((i % 8) * 16, 16)] = jnp.zeros((16,), jnp.float32)
      return carry

    lax.fori_loop(0, zr * 8, zero_body, 0)
    row0 = sid * rpt
    for j in range(rpt // zr):
      pltpu.sync_copy(zbuf, acc.at[pl.ds(row0 + j * zr, zr)])
    plsc.subcore_barrier()

    def body(i, carry):
      off = wid * epw + i * chunk
      pltpu.sync_copy(src_hbm.at[pl.ds(off, chunk)], sidx)
      pltpu.sync_copy(dst_hbm.at[pl.ds(off, chunk)], didx)
      pltpu.async_copy(h_hbm.at[sidx], rows, sem).wait()
      pltpu.sync_copy(rows, acc.at[didx], add=True)
      return carry

    lax.fori_loop(0, niter, body, 0)

    plsc.subcore_barrier()
    pltpu.sync_copy(acc.at[pl.ds(row0, rpt)],
                    out_hbm.at[cid, pl.ds(row0, rpt)])

  return segsum


@functools.cache
def _segsum_fn():
  return _make_segsum()


def _segsum(h, src, dst):
  return _segsum_fn()(h, src, dst)


def kernel(x, edge_index, lin_w, lin_b, conv_w1, conv_w2, bn_gamma, bn_beta):
  src = edge_index[0]
  dst = edge_index[1]
  lin_b2 = lin_b.reshape(1, F)
  g2 = bn_gamma.reshape(1, F)
  b2 = bn_beta.reshape(1, F)

  x0 = _compute_x0(x, lin_w, lin_b2)
  h = x0
  for layer, w in enumerate((conv_w1, conv_w2), start=1):
    beta = float(math.log(THETA / layer + 1.0))
    mpart = _segsum(h, src, dst)
    h = _dense_layer(mpart, x0, w, g2, b2, beta)
  return h
